# Initial kernel scaffold; baseline (speedup 1.0000x reference)
import jax
import jax.numpy as jnp
from jax.experimental import pallas as pl


def kernel(x, scope, W, bias):
    raise NotImplementedError("write your pallas kernel here")


# speedup vs baseline: 4.8308x; 4.8308x over previous
import functools

import jax
import jax.numpy as jnp
from jax import lax
from jax.experimental import pallas as pl
from jax.experimental.pallas import tpu as pltpu
from jax.experimental.pallas import tpu_sc as plsc

_N_TOKENS = 32768
_N_SEG = 4096
_D = 768
_NUM_CLASSES = 53
_C_PAD = 128

_NW = 32
_SEG_PER_W = _N_SEG // _NW
_SCOPE_CHUNK = 144
_LANES = 16
_D_CHUNKS = _D // _LANES


@functools.partial(
    pl.kernel,
    out_type=jax.ShapeDtypeStruct((_N_SEG, _D), jnp.float32),
    mesh=plsc.VectorSubcoreMesh(core_axis_name="c", subcore_axis_name="s"),
    scratch_types=[
        pltpu.VMEM((_SCOPE_CHUNK,), jnp.int32),
        pltpu.VMEM((_D,), jnp.float32),
        pltpu.VMEM((_D,), jnp.float32),
    ],
)
def _segment_mean(x_hbm, scope_hbm, out_hbm, scope_v, row_v, acc_v):
    wid = lax.axis_index("c") * 16 + lax.axis_index("s")
    seg0 = wid * _SEG_PER_W
    pltpu.sync_copy(scope_hbm.at[pl.ds(seg0, _SCOPE_CHUNK)], scope_v)

    def seg_body(i, _):
        start = scope_v[pl.ds(i, _LANES)][0]
        end = scope_v[pl.ds(i + 1, _LANES)][0]

        for k in range(_D_CHUNKS):
            acc_v[pl.ds(k * _LANES, _LANES)] = jnp.zeros((_LANES,), jnp.float32)

        def row_body(r, carry):
            pltpu.sync_copy(x_hbm.at[r], row_v)
            for k in range(_D_CHUNKS):
                sl = pl.ds(k * _LANES, _LANES)
                acc_v[sl] = acc_v[sl] + row_v[sl]
            return carry

        lax.fori_loop(start, end, row_body, 0)

        cnt = (end - start).astype(jnp.float32)
        cnt_vec = jnp.maximum(jnp.full((_LANES,), cnt, jnp.float32), 1.0)
        scale = jnp.full((_LANES,), 1.0, jnp.float32) / cnt_vec
        for k in range(_D_CHUNKS):
            sl = pl.ds(k * _LANES, _LANES)
            row_v[sl] = acc_v[sl] * scale
        pltpu.sync_copy(row_v, out_hbm.at[seg0 + i])
        return 0

    lax.fori_loop(0, _SEG_PER_W, seg_body, 0)


def _logits_body(repre_ref, w_ref, bias_ref, out_ref):
    out_ref[...] = lax.dot_general(
        repre_ref[...], w_ref[...],
        (((1,), (1,)), ((), ())),
        preferred_element_type=jnp.float32,
    ) + bias_ref[...]


_ROW_BLK = 512


@jax.jit
def _logits_matmul(repre, w_pad, bias_pad):
    return pl.pallas_call(
        _logits_body,
        grid=(_N_SEG // _ROW_BLK,),
        in_specs=[
            pl.BlockSpec((_ROW_BLK, _D), lambda i: (i, 0)),
            pl.BlockSpec((_C_PAD, _D), lambda i: (0, 0)),
            pl.BlockSpec((1, _C_PAD), lambda i: (0, 0)),
        ],
        out_specs=pl.BlockSpec((_ROW_BLK, _C_PAD), lambda i: (i, 0)),
        out_shape=jax.ShapeDtypeStruct((_N_SEG, _C_PAD), jnp.float32),
    )(repre, w_pad, bias_pad)


def kernel(x, scope, W, bias):
    scope_pad = jnp.concatenate(
        [scope, jnp.full((_NW * _SEG_PER_W + _SCOPE_CHUNK - scope.shape[0],),
                         _N_TOKENS, jnp.int32)])
    repre = _segment_mean(x, scope_pad)

    w_pad = jnp.zeros((_C_PAD, _D), jnp.float32).at[:_NUM_CLASSES].set(W)
    bias_pad = jnp.zeros((1, _C_PAD), jnp.float32).at[0, :_NUM_CLASSES].set(bias)
    logits = _logits_matmul(repre, w_pad, bias_pad)[:, :_NUM_CLASSES]

    w = jnp.transpose(W, (1, 0))
    return (repre, w, logits)


# speedup vs baseline: 33.7047x; 6.9771x over previous
import functools

import jax
import jax.numpy as jnp
from jax import lax
from jax.experimental import pallas as pl
from jax.experimental.pallas import tpu as pltpu
from jax.experimental.pallas import tpu_sc as plsc

_N_TOKENS = 32768
_N_SEG = 4096
_D = 768
_NUM_CLASSES = 53
_C_PAD = 128

_NW = 32
_SEG_PER_W = _N_SEG // _NW
_SCOPE_CHUNK = 160
_LANES = 16
_D_CHUNKS = _D // _LANES
_CROWS = 16
_NBUF = 4
_NOUT = 4


@functools.partial(
    pl.kernel,
    out_type=jax.ShapeDtypeStruct((_N_SEG, _D), jnp.float32),
    mesh=plsc.VectorSubcoreMesh(core_axis_name="c", subcore_axis_name="s"),
    scratch_types=[
        pltpu.VMEM((_SCOPE_CHUNK,), jnp.int32),
        pltpu.VMEM((_NBUF * _CROWS, _D), jnp.float32),
        pltpu.VMEM((_NOUT, _D), jnp.float32),
    ] + [pltpu.SemaphoreType.DMA] * (_NBUF + _NOUT),
)
def _segment_mean(x_hbm, scope_hbm, out_hbm, scope_v, bufs_v, orow_v,
                  *sems):
    in_sems = sems[:_NBUF]
    out_sems = sems[_NBUF:]
    wid = lax.axis_index("c") * 16 + lax.axis_index("s")
    seg0 = pl.multiple_of(wid * _SEG_PER_W, 128)
    pltpu.sync_copy(scope_hbm.at[pl.ds(seg0, _SCOPE_CHUNK)], scope_v)

    def bound(i):
        return scope_v[pl.ds(i, _LANES)][0]

    t0 = bound(0)
    t1 = bound(_SEG_PER_W)
    base = pl.multiple_of(t0 - t0 % _CROWS, _CROWS)

    def chunk_start(c):
        return pl.multiple_of(base + c * _CROWS, _CROWS)

    def issue(c, slot):
        @pl.when(jnp.logical_and(base + c * _CROWS < t1, t0 < t1))
        def _():
            pltpu.async_copy(
                x_hbm.at[pl.ds(chunk_start(c), _CROWS)],
                bufs_v.at[pl.ds(slot * _CROWS, _CROWS)], in_sems[slot])

    def on_crossing(c, slot):
        issue(c + 3, (slot + 3) % _NBUF)
        pltpu.make_async_copy(
            x_hbm.at[pl.ds(chunk_start(c), _CROWS)],
            bufs_v.at[pl.ds(slot * _CROWS, _CROWS)], in_sems[slot]).wait()

    for b in range(_NBUF - 1):
        issue(jnp.int32(b), b)

    zeros = tuple(jnp.zeros((_LANES,), jnp.float32) for _ in range(_D_CHUNKS))

    def quad_body(qi, carry):
        for q in range(_NOUT):
            i = qi * _NOUT + q
            start = bound(i)
            end = bound(i + 1)

            def row_body(r, accs):
                off = r - base
                c = off // _CROWS

                @pl.when(jnp.logical_or(off % _CROWS == 0, r == t0))
                def _():
                    cm = c % _NBUF
                    lax.cond(
                        cm < 2,
                        lambda: lax.cond(cm == 0,
                                         lambda: on_crossing(c, 0),
                                         lambda: on_crossing(c, 1)),
                        lambda: lax.cond(cm == 2,
                                         lambda: on_crossing(c, 2),
                                         lambda: on_crossing(c, 3)))

                idx = off % _CROWS + (c % _NBUF) * _CROWS
                return tuple(
                    accs[k] + bufs_v[idx, pl.ds(k * _LANES, _LANES)]
                    for k in range(_D_CHUNKS))

            accs = lax.fori_loop(start, end, row_body, zeros)

            @pl.when(qi > 0)
            def _():
                pltpu.make_async_copy(
                    orow_v.at[q], out_hbm.at[seg0 + i - _NOUT],
                    out_sems[q]).wait()

            cnt = (end - start).astype(jnp.float32)
            cnt_vec = jnp.maximum(jnp.full((_LANES,), cnt, jnp.float32), 1.0)
            scale = jnp.full((_LANES,), 1.0, jnp.float32) / cnt_vec
            for k in range(_D_CHUNKS):
                orow_v[q, pl.ds(k * _LANES, _LANES)] = accs[k] * scale
            pltpu.async_copy(orow_v.at[q], out_hbm.at[seg0 + i],
                             out_sems[q])
        return carry

    lax.fori_loop(0, _SEG_PER_W // _NOUT, quad_body, 0)

    for q in range(_NOUT):
        pltpu.make_async_copy(
            orow_v.at[q], out_hbm.at[seg0 + _SEG_PER_W - _NOUT + q],
            out_sems[q]).wait()


def _logits_body(repre_ref, w_ref, bias_ref, out_ref):
    out_ref[...] = lax.dot_general(
        repre_ref[...], w_ref[...],
        (((1,), (1,)), ((), ())),
        preferred_element_type=jnp.float32,
    ) + bias_ref[...]


_ROW_BLK = 512


@jax.jit
def _logits_matmul(repre, w_pad, bias_pad):
    return pl.pallas_call(
        _logits_body,
        grid=(_N_SEG // _ROW_BLK,),
        in_specs=[
            pl.BlockSpec((_ROW_BLK, _D), lambda i: (i, 0)),
            pl.BlockSpec((_C_PAD, _D), lambda i: (0, 0)),
            pl.BlockSpec((1, _C_PAD), lambda i: (0, 0)),
        ],
        out_specs=pl.BlockSpec((_ROW_BLK, _C_PAD), lambda i: (i, 0)),
        out_shape=jax.ShapeDtypeStruct((_N_SEG, _C_PAD), jnp.float32),
    )(repre, w_pad, bias_pad)


def kernel(x, scope, W, bias):
    scope_pad = jnp.concatenate(
        [scope, jnp.full((_NW * _SEG_PER_W + _SCOPE_CHUNK - scope.shape[0],),
                         _N_TOKENS, jnp.int32)])
    repre = _segment_mean(x, scope_pad)

    w_pad = jnp.zeros((_C_PAD, _D), jnp.float32).at[:_NUM_CLASSES].set(W)
    bias_pad = jnp.zeros((1, _C_PAD), jnp.float32).at[0, :_NUM_CLASSES].set(bias)
    logits = _logits_matmul(repre, w_pad, bias_pad)[:, :_NUM_CLASSES]

    w = jnp.transpose(W, (1, 0))
    return (repre, w, logits)


# speedup vs baseline: 33.7400x; 1.0010x over previous
import functools

import jax
import jax.numpy as jnp
from jax import lax
from jax.experimental import pallas as pl
from jax.experimental.pallas import tpu as pltpu
from jax.experimental.pallas import tpu_sc as plsc

_N_TOKENS = 32768
_N_SEG = 4096
_D = 768
_NUM_CLASSES = 53
_C_PAD = 128

_NW = 32
_SEG_PER_W = _N_SEG // _NW
_SCOPE_CHUNK = 160
_LANES = 16
_D_CHUNKS = _D // _LANES
_CROWS = 16
_NBUF = 4
_NOUT = 4


@functools.partial(
    pl.kernel,
    out_type=jax.ShapeDtypeStruct((_N_SEG, _D), jnp.float32),
    mesh=plsc.VectorSubcoreMesh(core_axis_name="c", subcore_axis_name="s"),
    scratch_types=[
        pltpu.VMEM((_SCOPE_CHUNK,), jnp.int32),
        pltpu.VMEM((_NBUF * _CROWS, _D), jnp.float32),
        pltpu.VMEM((_NOUT, _D), jnp.float32),
    ] + [pltpu.SemaphoreType.DMA] * (_NBUF + _NOUT),
)
def _segment_mean(x_hbm, scope_hbm, out_hbm, scope_v, bufs_v, orow_v,
                  *sems):
    in_sems = sems[:_NBUF]
    out_sems = sems[_NBUF:]
    wid = lax.axis_index("c") * 16 + lax.axis_index("s")
    seg0 = pl.multiple_of(wid * _SEG_PER_W, 128)
    pltpu.sync_copy(scope_hbm.at[pl.ds(seg0, _SCOPE_CHUNK)], scope_v)

    def bound(i):
        return scope_v[pl.ds(i, _LANES)][0]

    t0 = bound(0)
    t1 = bound(_SEG_PER_W)
    base = pl.multiple_of(t0 - t0 % _CROWS, _CROWS)

    def chunk_start(c):
        return pl.multiple_of(base + c * _CROWS, _CROWS)

    def issue(c, slot):
        @pl.when(jnp.logical_and(base + c * _CROWS < t1, t0 < t1))
        def _():
            pltpu.async_copy(
                x_hbm.at[pl.ds(chunk_start(c), _CROWS)],
                bufs_v.at[pl.ds(slot * _CROWS, _CROWS)], in_sems[slot])

    def on_crossing(c, slot):
        issue(c + 3, (slot + 3) % _NBUF)
        pltpu.make_async_copy(
            x_hbm.at[pl.ds(chunk_start(c), _CROWS)],
            bufs_v.at[pl.ds(slot * _CROWS, _CROWS)], in_sems[slot]).wait()

    for b in range(_NBUF - 1):
        issue(jnp.int32(b), b)

    zeros = tuple(jnp.zeros((_LANES,), jnp.float32)
                  for _ in range(_D_CHUNKS))

    def quad_body(qi, carry):
        for q in range(_NOUT):
            i = qi * _NOUT + q
            start = bound(i)
            end = bound(i + 1)

            def row_body(r, accs):
                off = r - base
                c = off // _CROWS

                @pl.when(jnp.logical_or(off % _CROWS == 0, r == t0))
                def _():
                    cm = c % _NBUF
                    lax.cond(
                        cm < 2,
                        lambda: lax.cond(cm == 0,
                                         lambda: on_crossing(c, 0),
                                         lambda: on_crossing(c, 1)),
                        lambda: lax.cond(cm == 2,
                                         lambda: on_crossing(c, 2),
                                         lambda: on_crossing(c, 3)))

                idx = off % _CROWS + (c % _NBUF) * _CROWS
                return tuple(
                    accs[k] + bufs_v[idx, pl.ds(k * _LANES, _LANES)]
                    for k in range(_D_CHUNKS))

            accs = lax.fori_loop(start, end, row_body, zeros)

            @pl.when(qi > 0)
            def _():
                pltpu.make_async_copy(
                    orow_v.at[q], out_hbm.at[seg0 + i - _NOUT],
                    out_sems[q]).wait()

            cnt = (end - start).astype(jnp.float32)
            cnt_vec = jnp.maximum(jnp.full((_LANES,), cnt, jnp.float32), 1.0)
            scale = jnp.full((_LANES,), 1.0, jnp.float32) / cnt_vec
            for k in range(_D_CHUNKS):
                orow_v[q, pl.ds(k * _LANES, _LANES)] = accs[k] * scale
            pltpu.async_copy(orow_v.at[q], out_hbm.at[seg0 + i],
                             out_sems[q])
        return carry

    lax.fori_loop(0, _SEG_PER_W // _NOUT, quad_body, 0)

    for q in range(_NOUT):
        pltpu.make_async_copy(
            orow_v.at[q], out_hbm.at[seg0 + _SEG_PER_W - _NOUT + q],
            out_sems[q]).wait()


def _logits_body(repre_ref, w_ref, bias_ref, out_ref):
    out_ref[...] = lax.dot_general(
        repre_ref[...], w_ref[...],
        (((1,), (1,)), ((), ())),
        preferred_element_type=jnp.float32,
    ) + bias_ref[...]


_ROW_BLK = 512


@jax.jit
def _logits_matmul(repre, W, bias2d):
    return pl.pallas_call(
        _logits_body,
        grid=(_N_SEG // _ROW_BLK,),
        in_specs=[
            pl.BlockSpec((_ROW_BLK, _D), lambda i: (i, 0)),
            pl.BlockSpec((_NUM_CLASSES, _D), lambda i: (0, 0)),
            pl.BlockSpec((1, _NUM_CLASSES), lambda i: (0, 0)),
        ],
        out_specs=pl.BlockSpec((_ROW_BLK, _NUM_CLASSES), lambda i: (i, 0)),
        out_shape=jax.ShapeDtypeStruct((_N_SEG, _NUM_CLASSES), jnp.float32),
    )(repre, W, bias2d)


def kernel(x, scope, W, bias):
    scope_pad = jnp.concatenate(
        [scope, jnp.full((_NW * _SEG_PER_W + _SCOPE_CHUNK - scope.shape[0],),
                         _N_TOKENS, jnp.int32)])
    repre = _segment_mean(x, scope_pad)

    logits = _logits_matmul(repre, W, bias.reshape(1, _NUM_CLASSES))

    w = jnp.transpose(W, (1, 0))
    return (repre, w, logits)
